# Initial kernel scaffold; baseline (speedup 1.0000x reference)
import functools

import jax
import jax.numpy as jnp
from jax import lax
from jax.experimental import pallas as pl
from jax.experimental.pallas import tpu as pltpu
from jax.experimental.pallas import tpu_sc as plsc

_N_FIELDS = 26
_VOCAB = 100000
_EMBED_DIM = 32
_BATCH = 16384

_NC = 2
_NS = 16
_NW = _NC * _NS
_LANES = 16

_N_ROWS = _BATCH * _N_FIELDS
_PER_W = _N_ROWS // _NW
_R = 128
_G = 8
_CHUNK = _G * _R
_NCHUNK = _PER_W // _CHUNK


def _sc_gather(idx, table_flat):
    mesh = plsc.VectorSubcoreMesh(core_axis_name="c", subcore_axis_name="s")

    @functools.partial(
        pl.kernel,
        mesh=mesh,
        out_type=jax.ShapeDtypeStruct((_NW, _NCHUNK, _G, _R, _EMBED_DIM),
                                      jnp.float32),
        scratch_types=[
            pltpu.VMEM((_NCHUNK, _G, _R), jnp.int32),
            pltpu.VMEM((_G, _R, _EMBED_DIM), jnp.float32),
            pltpu.SemaphoreType.DMA,
        ],
    )
    def k(idx_hbm, tab_hbm, out_hbm, idx_v, rows_v, gsem):
        wid = lax.axis_index("s") * _NC + lax.axis_index("c")
        pltpu.sync_copy(idx_hbm.at[wid], idx_v)
        lanes = lax.iota(jnp.int32, 16)

        def do_chunk(c, carry):
            for g in range(_G):
                for l in range(_R // _LANES):
                    q0 = c * _CHUNK + g * _R + l * _LANES
                    f = lax.rem(q0 + lanes, _N_FIELDS)
                    vec = idx_v[c, g, pl.ds(l * _LANES, _LANES)]
                    idx_v[c, g, pl.ds(l * _LANES, _LANES)] = (
                        vec + f * _VOCAB)
            cps = [
                pltpu.async_copy(tab_hbm.at[idx_v.at[c, g]], rows_v.at[g],
                                 gsem)
                for g in range(_G)
            ]
            for cp in cps:
                cp.wait()
            pltpu.sync_copy(rows_v, out_hbm.at[wid, c])
            return carry

        lax.fori_loop(0, _NCHUNK, do_chunk, 0)

    return k(idx, table_flat)


def kernel(inputs, tables):
    idx = inputs.astype(jnp.int32).reshape(_NW, _NCHUNK, _G, _R)
    table_flat = tables.reshape(_N_FIELDS * _VOCAB, _EMBED_DIM)
    out = _sc_gather(idx, table_flat)
    return out.reshape(_BATCH, _N_FIELDS * _EMBED_DIM)


# speedup vs baseline: 1.2076x; 1.2076x over previous
import functools

import jax
import jax.numpy as jnp
from jax import lax
from jax.experimental import pallas as pl
from jax.experimental.pallas import tpu as pltpu
from jax.experimental.pallas import tpu_sc as plsc

_N_FIELDS = 26
_VOCAB = 100000
_EMBED_DIM = 32
_BATCH = 16384

_NC = 2
_NS = 16
_NW = _NC * _NS
_LANES = 16

_N_ROWS = _BATCH * _N_FIELDS
_PER_W = _N_ROWS // _NW
_R = 128
_G = 8
_CHUNK = _G * _R
_NCHUNK = _PER_W // _CHUNK


def _sc_gather(idx, table_flat):
    mesh = plsc.VectorSubcoreMesh(core_axis_name="c", subcore_axis_name="s")

    @functools.partial(
        pl.kernel,
        mesh=mesh,
        out_type=jax.ShapeDtypeStruct((_NW, _NCHUNK, _G, _R, _EMBED_DIM),
                                      jnp.float32),
        scratch_types=[
            pltpu.VMEM((_NCHUNK, _G, _R), jnp.int32),
            pltpu.VMEM((_G, _R, _EMBED_DIM), jnp.float32),
            pltpu.SemaphoreType.DMA,
        ],
        compiler_params=pltpu.CompilerParams(use_tc_tiling_on_sc=False),
    )
    def k(idx_hbm, tab_hbm, out_hbm, idx_v, rows_v, gsem):
        wid = lax.axis_index("s") * _NC + lax.axis_index("c")
        pltpu.sync_copy(idx_hbm.at[wid], idx_v)
        lanes = lax.iota(jnp.int32, 16)

        def do_chunk(c, carry):
            for g in range(_G):
                for l in range(_R // _LANES):
                    q0 = c * _CHUNK + g * _R + l * _LANES
                    f = lax.rem(q0 + lanes, _N_FIELDS)
                    vec = idx_v[c, g, pl.ds(l * _LANES, _LANES)]
                    idx_v[c, g, pl.ds(l * _LANES, _LANES)] = (
                        vec + f * _VOCAB)
            cps = [
                pltpu.async_copy(tab_hbm.at[idx_v.at[c, g]], rows_v.at[g],
                                 gsem)
                for g in range(_G)
            ]
            for cp in cps:
                cp.wait()
            pltpu.sync_copy(rows_v, out_hbm.at[wid, c])
            return carry

        lax.fori_loop(0, _NCHUNK, do_chunk, 0)

    return k(idx, table_flat)


def kernel(inputs, tables):
    idx = inputs.astype(jnp.int32).reshape(_NW, _NCHUNK, _G, _R)
    table_flat = tables.reshape(_N_FIELDS * _VOCAB, _EMBED_DIM)
    out = _sc_gather(idx, table_flat)
    return out.reshape(_BATCH, _N_FIELDS * _EMBED_DIM)


# speedup vs baseline: 1.2170x; 1.0077x over previous
import functools

import jax
import jax.numpy as jnp
from jax import lax
from jax.experimental import pallas as pl
from jax.experimental.pallas import tpu as pltpu
from jax.experimental.pallas import tpu_sc as plsc

_N_FIELDS = 26
_VOCAB = 100000
_EMBED_DIM = 32
_BATCH = 16384

_NC = 2
_NS = 16
_NW = _NC * _NS
_LANES = 16

_N_ROWS = _BATCH * _N_FIELDS
_PER_W = _N_ROWS // _NW
_R = 128
_G = 8
_CHUNK = _G * _R
_NCHUNK = _PER_W // _CHUNK


def _sc_gather(idx, table_flat):
    mesh = plsc.VectorSubcoreMesh(core_axis_name="c", subcore_axis_name="s")

    @functools.partial(
        pl.kernel,
        mesh=mesh,
        out_type=jax.ShapeDtypeStruct((_NW, _NCHUNK, _G, _R, _EMBED_DIM),
                                      jnp.float32),
        scratch_types=[
            pltpu.VMEM((_NCHUNK, _G, _R), jnp.int32),
            pltpu.VMEM((2, _G, _R, _EMBED_DIM), jnp.float32),
            pltpu.SemaphoreType.DMA,
            pltpu.SemaphoreType.DMA,
        ],
        compiler_params=pltpu.CompilerParams(use_tc_tiling_on_sc=False),
    )
    def k(idx_hbm, tab_hbm, out_hbm, idx_v, rows_v, gsem0, gsem1):
        wid = lax.axis_index("s") * _NC + lax.axis_index("c")
        gsems = [gsem0, gsem1]
        pltpu.sync_copy(idx_hbm.at[wid], idx_v)
        lanes = lax.iota(jnp.int32, 16)

        def adjust(c):
            for g in range(_G):
                for l in range(_R // _LANES):
                    q0 = c * _CHUNK + g * _R + l * _LANES
                    f = lax.rem(q0 + lanes, _N_FIELDS)
                    vec = idx_v[c, g, pl.ds(l * _LANES, _LANES)]
                    idx_v[c, g, pl.ds(l * _LANES, _LANES)] = (
                        vec + f * _VOCAB)

        def fire(c, b):
            for g in range(_G):
                pltpu.async_copy(tab_hbm.at[idx_v.at[c, g]], rows_v.at[b, g],
                                 gsems[b])

        def drain(b):
            pltpu.make_async_copy(out_hbm.at[wid, 0], rows_v.at[b],
                                  gsems[b]).wait()

        adjust(0)
        fire(0, 0)

        def pair(cc, carry):
            for d in range(2):
                c = cc + d
                b = d
                adjust(c + 1)
                fire(c + 1, 1 - b)
                drain(b)
                pltpu.sync_copy(rows_v.at[b], out_hbm.at[wid, c])
            return carry

        lax.fori_loop(0, _NCHUNK // 2, lambda i, u: pair(2 * i, u), 0,
                      unroll=False)
        drain(0)
        pltpu.sync_copy(rows_v.at[0], out_hbm.at[wid, _NCHUNK - 1])

    return k(idx, table_flat)


def kernel(inputs, tables):
    idx = inputs.astype(jnp.int32).reshape(_NW, _NCHUNK, _G, _R)
    table_flat = tables.reshape(_N_FIELDS * _VOCAB, _EMBED_DIM)
    out = _sc_gather(idx, table_flat)
    return out.reshape(_BATCH, _N_FIELDS * _EMBED_DIM)
